# baseline (speedup 1.0000x reference)
import functools

import jax
import jax.numpy as jnp
from jax import lax
from jax.experimental import pallas as pl
from jax.experimental.pallas import tpu as pltpu
from jax.experimental.pallas import tpu_sc as plsc


def _make_sc_kernel(batch, seq_len, d_model, dtype):
    info = plsc.get_sparse_core_info()
    nc, ns = info.num_cores, info.num_subcores
    nw = nc * ns
    rows_per_w = seq_len // nw
    mesh = plsc.VectorSubcoreMesh(core_axis_name="c", subcore_axis_name="s")

    @functools.partial(
        pl.kernel,
        mesh=mesh,
        out_type=jax.ShapeDtypeStruct((batch, seq_len, d_model), dtype),
        scratch_types=[
            pltpu.VMEM((rows_per_w, d_model), dtype),
            pltpu.SemaphoreType.DMA((batch,)),
        ],
    )
    def sc_broadcast(w_hbm, out_hbm, rows_v, sems):
        wid = lax.axis_index("s") * nc + lax.axis_index("c")
        base = wid * rows_per_w
        pltpu.sync_copy(w_hbm.at[pl.ds(base, rows_per_w), :], rows_v)
        cps = []
        for b in range(batch):
            cp = pltpu.make_async_copy(
                rows_v, out_hbm.at[b, pl.ds(base, rows_per_w), :], sems.at[b]
            )
            cp.start()
            cps.append(cp)
        for cp in cps:
            cp.wait()

    return sc_broadcast


def kernel(tokens, W_pos):
    batch, seq_len = tokens.shape
    d_model = W_pos.shape[1]
    sc = _make_sc_kernel(batch, seq_len, d_model, W_pos.dtype)
    return sc(W_pos[:seq_len])


# speedup vs baseline: 2.3224x; 2.3224x over previous
import jax
import jax.numpy as jnp
from jax.experimental import pallas as pl
from jax.experimental.pallas import tpu as pltpu


_CHUNKS = 2


def _copy_body(w_ref, out_ref, vmem, in_sems, out_sems):
    batch = out_ref.shape[0]
    seq_len = w_ref.shape[0]
    chunk = seq_len // _CHUNKS
    ins = []
    for c in range(_CHUNKS):
        sl = pl.ds(c * chunk, chunk)
        cp = pltpu.make_async_copy(w_ref.at[sl, :], vmem.at[sl, :], in_sems.at[c])
        cp.start()
        ins.append(cp)
    outs = []
    for c in range(_CHUNKS):
        ins[c].wait()
        sl = pl.ds(c * chunk, chunk)
        for b in range(batch):
            cp = pltpu.make_async_copy(
                vmem.at[sl, :], out_ref.at[b, sl, :], out_sems.at[b, c]
            )
            cp.start()
            outs.append(cp)
    for cp in outs:
        cp.wait()


def kernel(tokens, W_pos):
    batch, seq_len = tokens.shape
    d_model = W_pos.shape[1]
    out = pl.pallas_call(
        _copy_body,
        in_specs=[pl.BlockSpec(memory_space=pl.ANY)],
        out_specs=pl.BlockSpec(memory_space=pl.ANY),
        out_shape=jax.ShapeDtypeStruct((batch, seq_len, d_model), W_pos.dtype),
        scratch_shapes=[
            pltpu.VMEM((seq_len, d_model), W_pos.dtype),
            pltpu.SemaphoreType.DMA((_CHUNKS,)),
            pltpu.SemaphoreType.DMA((batch, _CHUNKS)),
        ],
    )(W_pos[:seq_len])
    return out


# speedup vs baseline: 2.3910x; 1.0295x over previous
import jax
import jax.numpy as jnp
from jax.experimental import pallas as pl
from jax.experimental.pallas import tpu as pltpu


_CHUNKS = 16


def _copy_body(w_ref, out_ref, vmem, in_sems, out_sems):
    batch = out_ref.shape[0]
    seq_len = w_ref.shape[0]
    chunk = seq_len // _CHUNKS
    ins = []
    for c in range(_CHUNKS):
        sl = pl.ds(c * chunk, chunk)
        cp = pltpu.make_async_copy(w_ref.at[sl, :], vmem.at[sl, :], in_sems.at[c])
        cp.start()
        ins.append(cp)
    outs = []
    for c in range(_CHUNKS):
        ins[c].wait()
        sl = pl.ds(c * chunk, chunk)
        for b in range(batch):
            cp = pltpu.make_async_copy(
                vmem.at[sl, :], out_ref.at[b, sl, :], out_sems.at[b, c]
            )
            cp.start()
            outs.append(cp)
    for cp in outs:
        cp.wait()


def kernel(tokens, W_pos):
    batch, seq_len = tokens.shape
    d_model = W_pos.shape[1]
    out = pl.pallas_call(
        _copy_body,
        in_specs=[pl.BlockSpec(memory_space=pl.ANY)],
        out_specs=pl.BlockSpec(memory_space=pl.ANY),
        out_shape=jax.ShapeDtypeStruct((batch, seq_len, d_model), W_pos.dtype),
        scratch_shapes=[
            pltpu.VMEM((seq_len, d_model), W_pos.dtype),
            pltpu.SemaphoreType.DMA((_CHUNKS,)),
            pltpu.SemaphoreType.DMA((batch, _CHUNKS)),
        ],
    )(W_pos[:seq_len])
    return out
